# baseline (speedup 1.0000x reference)
import functools

import jax
import jax.numpy as jnp
from jax import lax
from jax.experimental import pallas as pl
from jax.experimental.pallas import tpu as pltpu
from jax.experimental.pallas import tpu_sc as plsc

B = 4096
DIM = 64
N_OBJ = 17
N_CTX = 16
N_ENT_ROWS = B * 33
N_REL_ROWS = B * 16

NW = 32
CH = 128
ENT_CHUNKS = N_ENT_ROWS // (NW * CH)
REL_CHUNKS = N_REL_ROWS // (NW * CH)

BB = 256
GRID = B // BB


def _sc_gather_body(ent_emb, rel_emb, ent_idx, rel_idx, out_ent, out_rel,
                    idx_ent_v, idx_rel_v, rows_v, sem):
    wid = lax.axis_index("s") * 2 + lax.axis_index("c")
    pltpu.sync_copy(ent_idx.at[wid], idx_ent_v)
    pltpu.sync_copy(rel_idx.at[wid], idx_rel_v)

    ebase = wid * (ENT_CHUNKS * CH)
    rbase = wid * (REL_CHUNKS * CH)

    def ent_step(j, _):
        pltpu.async_copy(ent_emb.at[idx_ent_v.at[j]], rows_v, sem).wait()
        pltpu.sync_copy(rows_v, out_ent.at[pl.ds(ebase + j * CH, CH)])
        return 0

    lax.fori_loop(0, ENT_CHUNKS, ent_step, 0)

    def rel_step(j, _):
        pltpu.async_copy(rel_emb.at[idx_rel_v.at[j]], rows_v, sem).wait()
        pltpu.sync_copy(rows_v, out_rel.at[pl.ds(rbase + j * CH, CH)])
        return 0

    lax.fori_loop(0, REL_CHUNKS, rel_step, 0)


_sc_gather = functools.partial(
    pl.kernel,
    out_type=(
        jax.ShapeDtypeStruct((N_ENT_ROWS, DIM), jnp.float32),
        jax.ShapeDtypeStruct((N_REL_ROWS, DIM), jnp.float32),
    ),
    mesh=plsc.VectorSubcoreMesh(core_axis_name="c", subcore_axis_name="s"),
    scratch_types=(
        pltpu.VMEM((ENT_CHUNKS, CH), jnp.int32),
        pltpu.VMEM((REL_CHUNKS, CH), jnp.int32),
        pltpu.VMEM((CH, DIM), jnp.float32),
        pltpu.SemaphoreType.DMA,
    ),
    compiler_params=pltpu.CompilerParams(use_tc_tiling_on_sc=False),
)(_sc_gather_body)


def _tc_body(ent_ref, rel_ref, out_ref):
    e = ent_ref[...]
    r = rel_ref[...]
    obj = e[:, :N_OBJ, :]
    hh = e[:, 17:25, :] + r[:, :8, :]
    tt = e[:, 25:33, :] - r[:, 8:16, :]
    ctx = jnp.concatenate([hh, tt], axis=1)
    acc = jnp.zeros((BB, N_OBJ), jnp.float32)
    for k in range(N_CTX):
        d = obj - ctx[:, k:k + 1, :]
        acc = acc + jnp.sqrt(jnp.sum(d * d, axis=2))
    f1 = acc * (-1.0 / N_CTX)
    col = lax.broadcasted_iota(jnp.int32, (BB, N_OBJ), 1)
    x = jnp.where(col == 0, f1, -f1)
    ls = jnp.minimum(x, 0.0) - jnp.log1p(jnp.exp(-jnp.abs(x)))
    bs = jnp.sum(ls)

    @pl.when(pl.program_id(0) == 0)
    def _():
        out_ref[0, 0] = 0.0

    out_ref[0, 0] += bs


_tc_reduce = pl.pallas_call(
    _tc_body,
    grid=(GRID,),
    in_specs=[
        pl.BlockSpec((BB, 33, DIM), lambda i: (i, 0, 0)),
        pl.BlockSpec((BB, 16, DIM), lambda i: (i, 0, 0)),
    ],
    out_specs=pl.BlockSpec((1, 1), lambda i: (0, 0),
                           memory_space=pltpu.SMEM),
    out_shape=jax.ShapeDtypeStruct((1, 1), jnp.float32),
    compiler_params=pltpu.CompilerParams(
        dimension_semantics=("arbitrary",)),
)


def kernel(entity_batch, head_batch, head_relation_batch,
           tail_relation_batch, tail_batch, negative_batch,
           entity_emb, relation_emb):
    obj_idx = jnp.concatenate([entity_batch[:, None], negative_batch], axis=1)
    ent_idx = jnp.concatenate([obj_idx, head_batch, tail_batch], axis=1)
    ent_idx = ent_idx.astype(jnp.int32).reshape(NW, ENT_CHUNKS, CH)
    rel_idx = jnp.concatenate([head_relation_batch, tail_relation_batch],
                              axis=1)
    rel_idx = rel_idx.astype(jnp.int32).reshape(NW, REL_CHUNKS, CH)

    out_ent, out_rel = _sc_gather(entity_emb, relation_emb, ent_idx, rel_idx)
    total = _tc_reduce(out_ent.reshape(B, 33, DIM),
                       out_rel.reshape(B, 16, DIM))
    return total[0, 0]


# speedup vs baseline: 1.2745x; 1.2745x over previous
import functools

import jax
import jax.numpy as jnp
from jax import lax
from jax.experimental import pallas as pl
from jax.experimental.pallas import tpu as pltpu
from jax.experimental.pallas import tpu_sc as plsc

B = 4096
DIM = 64
N_OBJ = 17
N_CTX = 16
N_ENT_ROWS = B * 33
N_REL_ROWS = B * 16

NW = 32
EPW = N_ENT_ROWS // NW
RPW = N_REL_ROWS // NW
ECH = 528
RCH = 512

BB = 256
GRID = B // BB


def _gather_chunk(table, idx_hbm, out_hbm, idx_v, rows_v, sem, src_base,
                  dst_base, n, lanes):
    pltpu.sync_copy(idx_hbm.at[pl.ds(src_base, n)], idx_v.at[pl.ds(0, n)])

    def vec(v16, _):
        iv = idx_v[pl.ds(v16 * 16, 16)]

        def row(l, _):
            r = jnp.sum(jnp.where(lanes == l, iv, 0))
            pltpu.async_copy(table.at[pl.ds(r, 1)],
                             rows_v.at[pl.ds(v16 * 16 + l, 1)], sem)
            return 0

        lax.fori_loop(0, 16, row, 0)
        return 0

    lax.fori_loop(0, n // 16, vec, 0)

    def drain(i, _):
        pltpu.make_async_copy(table.at[pl.ds(0, 1)],
                              rows_v.at[pl.ds(i, 1)], sem).wait()
        return 0

    lax.fori_loop(0, n, drain, 0)
    pltpu.sync_copy(rows_v.at[pl.ds(0, n)], out_hbm.at[pl.ds(dst_base, n)])


def _sc_body(ent_emb, rel_emb, ent_idx, rel_idx, out_ent, out_rel,
             idx_v, rows_v, sem):
    wid = lax.axis_index("s") * 2 + lax.axis_index("c")
    lanes = lax.broadcasted_iota(jnp.int32, (16,), 0)
    ebase = wid * EPW
    rbase = wid * RPW

    def ent_chunk(c, _):
        _gather_chunk(ent_emb, ent_idx, out_ent, idx_v, rows_v, sem,
                      ebase + c * ECH, ebase + c * ECH, ECH, lanes)
        return 0

    lax.fori_loop(0, EPW // ECH, ent_chunk, 0)

    def rel_chunk(c, _):
        _gather_chunk(rel_emb, rel_idx, out_rel, idx_v, rows_v, sem,
                      rbase + c * RCH, rbase + c * RCH, RCH, lanes)
        return 0

    lax.fori_loop(0, RPW // RCH, rel_chunk, 0)


_sc_gather = functools.partial(
    pl.kernel,
    out_type=(
        jax.ShapeDtypeStruct((N_ENT_ROWS, DIM), jnp.float32),
        jax.ShapeDtypeStruct((N_REL_ROWS, DIM), jnp.float32),
    ),
    mesh=plsc.VectorSubcoreMesh(core_axis_name="c", subcore_axis_name="s"),
    scratch_types=(
        pltpu.VMEM((ECH,), jnp.int32),
        pltpu.VMEM((ECH, DIM), jnp.float32),
        pltpu.SemaphoreType.DMA,
    ),
    compiler_params=pltpu.CompilerParams(needs_layout_passes=False),
)(_sc_body)


def _tc_body(ent_ref, rel_ref, out_ref):
    e = ent_ref[...]
    r = rel_ref[...]
    obj = e[:, :N_OBJ, :]
    hh = e[:, 17:25, :] + r[:, :8, :]
    tt = e[:, 25:33, :] - r[:, 8:16, :]
    ctx = jnp.concatenate([hh, tt], axis=1)
    acc = jnp.zeros((BB, N_OBJ), jnp.float32)
    for k in range(N_CTX):
        d = obj - ctx[:, k:k + 1, :]
        acc = acc + jnp.sqrt(jnp.sum(d * d, axis=2))
    f1 = acc * (-1.0 / N_CTX)
    col = lax.broadcasted_iota(jnp.int32, (BB, N_OBJ), 1)
    x = jnp.where(col == 0, f1, -f1)
    ls = jnp.minimum(x, 0.0) - jnp.log1p(jnp.exp(-jnp.abs(x)))
    bs = jnp.sum(ls)

    @pl.when(pl.program_id(0) == 0)
    def _():
        out_ref[0, 0] = 0.0

    out_ref[0, 0] += bs


_tc_reduce = pl.pallas_call(
    _tc_body,
    grid=(GRID,),
    in_specs=[
        pl.BlockSpec((BB, 33, DIM), lambda i: (i, 0, 0)),
        pl.BlockSpec((BB, 16, DIM), lambda i: (i, 0, 0)),
    ],
    out_specs=pl.BlockSpec((1, 1), lambda i: (0, 0),
                           memory_space=pltpu.SMEM),
    out_shape=jax.ShapeDtypeStruct((1, 1), jnp.float32),
    compiler_params=pltpu.CompilerParams(
        dimension_semantics=("arbitrary",)),
)


def kernel(entity_batch, head_batch, head_relation_batch,
           tail_relation_batch, tail_batch, negative_batch,
           entity_emb, relation_emb):
    obj_idx = jnp.concatenate([entity_batch[:, None], negative_batch], axis=1)
    ent_idx = jnp.concatenate([obj_idx, head_batch, tail_batch], axis=1)
    ent_idx = ent_idx.astype(jnp.int32).reshape(-1)
    rel_idx = jnp.concatenate([head_relation_batch, tail_relation_batch],
                              axis=1)
    rel_idx = rel_idx.astype(jnp.int32).reshape(-1)

    out_ent, out_rel = _sc_gather(entity_emb, relation_emb, ent_idx, rel_idx)
    total = _tc_reduce(out_ent.reshape(B, 33, DIM),
                       out_rel.reshape(B, 16, DIM))
    return total[0, 0]
